# Initial kernel scaffold; baseline (speedup 1.0000x reference)
import jax
import jax.numpy as jnp
from jax import lax
from jax.experimental import pallas as pl
from jax.experimental.pallas import tpu as pltpu
from jax.experimental.pallas import tpu_sc as plsc

NUM_SEG = 1024
ACC_W = 80
NC, NS = 2, 16
NW = NC * NS
CHUNK = 128
BK = 4000


def _gate_body(x_ref, w1_ref, b1_ref, w2_ref, b2_ref, g_ref, gmax_ref):
    i = pl.program_id(0)
    h = jnp.maximum(x_ref[...] @ w1_ref[...] + b1_ref[...], 0.0)
    g = h @ w2_ref[...] + b2_ref[...]
    g_ref[...] = g
    bm = jnp.max(g)

    @pl.when(i == 0)
    def _():
        gmax_ref[0, 0] = bm

    @pl.when(i > 0)
    def _():
        gmax_ref[0, 0] = jnp.maximum(gmax_ref[0, 0], bm)


def _gate(x, gW1, gb1, gW2, gb2):
    n = x.shape[0]
    return pl.pallas_call(
        _gate_body,
        grid=(n // BK,),
        in_specs=[
            pl.BlockSpec((BK, 64), lambda i: (i, 0)),
            pl.BlockSpec((64, 32), lambda i: (0, 0)),
            pl.BlockSpec((1, 32), lambda i: (0, 0)),
            pl.BlockSpec((32, 1), lambda i: (0, 0)),
            pl.BlockSpec((1, 1), lambda i: (0, 0)),
        ],
        out_specs=[
            pl.BlockSpec((BK, 1), lambda i: (i, 0)),
            pl.BlockSpec((1, 1), lambda i: (0, 0)),
        ],
        out_shape=[
            jax.ShapeDtypeStruct((n, 1), jnp.float32),
            jax.ShapeDtypeStruct((1, 1), jnp.float32),
        ],
    )(x, gW1, gb1, gW2, gb2)


def _sc_pool(x, gpad, spad, gmax16):
    n = x.shape[0]
    rows_per_tile = n // NW
    nchunk = rows_per_tile // CHUNK
    tail = rows_per_tile - nchunk * CHUNK
    tailpad = -(-tail // 16) * 16 if tail else 0

    mesh = plsc.VectorSubcoreMesh(
        core_axis_name="c", subcore_axis_name="s", num_cores=NC, num_subcores=NS
    )

    def body(x_hbm, g_hbm, s_hbm, gmax_hbm, out_hbm, xbuf, gbuf, sbuf, gmv, acc):
        wid = lax.axis_index("c") * NS + lax.axis_index("s")
        base = wid * rows_per_tile
        iota = lax.iota(jnp.int32, 16)
        onehot0 = (iota == 0).astype(jnp.float32)
        zeros = jnp.zeros((16,), jnp.float32)

        pltpu.sync_copy(gmax_hbm, gmv)

        def zbody(i, _):
            for d in range(ACC_W // 16):
                acc[i, pl.ds(16 * d, 16)] = zeros
            return 0

        lax.fori_loop(0, NUM_SEG, zbody, 0)

        def group(g0, nrows):
            seg16 = sbuf[pl.ds(g0, 16)]
            e16 = jnp.exp(gbuf[pl.ds(g0, 16)] - gmv[...])
            for r in range(nrows):
                idxr = jnp.full((16,), r, jnp.int32)
                e_b = jnp.take(e16, idxr, mode="promise_in_bounds")
                s_b = jnp.take(seg16, idxr, mode="promise_in_bounds")
                for d in range(4):
                    xv = xbuf[g0 + r, pl.ds(16 * d, 16)]
                    plsc.addupdate_scatter(acc, [s_b, iota + 16 * d], xv * e_b)
                plsc.addupdate_scatter(acc, [s_b, iota + 64], e_b * onehot0)

        def chunk_body(c, _):
            row0 = base + c * CHUNK
            pltpu.sync_copy(x_hbm.at[pl.ds(row0, CHUNK)], xbuf)
            pltpu.sync_copy(g_hbm.at[pl.ds(row0, CHUNK)], gbuf)
            pltpu.sync_copy(s_hbm.at[pl.ds(row0, CHUNK)], sbuf)
            for g0 in range(0, CHUNK, 16):
                group(g0, 16)
            return 0

        lax.fori_loop(0, nchunk, chunk_body, 0)

        if tail:
            row0 = base + nchunk * CHUNK
            pltpu.sync_copy(x_hbm.at[pl.ds(row0, tail)], xbuf.at[pl.ds(0, tail)])
            pltpu.sync_copy(g_hbm.at[pl.ds(row0, tailpad)], gbuf.at[pl.ds(0, tailpad)])
            pltpu.sync_copy(s_hbm.at[pl.ds(row0, tailpad)], sbuf.at[pl.ds(0, tailpad)])
            for g0 in range(0, tail, 16):
                group(g0, min(16, tail - g0))

        pltpu.sync_copy(acc, out_hbm.at[wid])

    run = pl.kernel(
        body,
        out_type=jax.ShapeDtypeStruct((NW, NUM_SEG, ACC_W), jnp.float32),
        mesh=mesh,
        scratch_types=[
            pltpu.VMEM((CHUNK, 64), jnp.float32),
            pltpu.VMEM((CHUNK,), jnp.float32),
            pltpu.VMEM((CHUNK,), jnp.int32),
            pltpu.VMEM((16,), jnp.float32),
            pltpu.VMEM((NUM_SEG, ACC_W), jnp.float32),
        ],
    )
    return run(x, gpad, spad, gmax16)


def _final_body(p_ref, w1_ref, b1_ref, w2_ref, b2_ref, out_ref):
    s = jnp.sum(p_ref[...], axis=0)
    num = s[:, :64]
    den = s[:, 64:65]
    hg = num / (den + 1e-16)
    h = jnp.maximum(hg @ w1_ref[...] + b1_ref[...], 0.0)
    out_ref[...] = h @ w2_ref[...] + b2_ref[...]


def _final(partials, mW1, mb1, mW2, mb2):
    return pl.pallas_call(
        _final_body,
        out_shape=jax.ShapeDtypeStruct((NUM_SEG, 1), jnp.float32),
    )(partials, mW1, mb1, mW2, mb2)


@jax.jit
def kernel(x, batch, gW1, gb1, gW2, gb2, mW1, mb1, mW2, mb2):
    g2d, gmax = _gate(x, gW1, gb1.reshape(1, -1), gW2, gb2.reshape(1, -1))
    g = g2d.reshape(-1)
    seg = batch.astype(jnp.int32)
    gpad = jnp.concatenate([g, jnp.zeros((16,), jnp.float32)])
    spad = jnp.concatenate([seg, jnp.zeros((16,), jnp.int32)])
    gmax16 = jnp.broadcast_to(gmax.reshape(1), (16,))
    partials = _sc_pool(x, gpad, spad, gmax16)
    return _final(partials, mW1, mb1.reshape(1, -1), mW2, mb2.reshape(1, -1))


# speedup vs baseline: 5.8151x; 5.8151x over previous
import jax
import jax.numpy as jnp
from jax import lax
from jax.experimental import pallas as pl
from jax.experimental.pallas import tpu as pltpu
from jax.experimental.pallas import tpu_sc as plsc

NUM_SEG = 1024
ACC_W = 80
NC, NS = 2, 16
NW = NC * NS
CHUNK = 128
BK = 4000


def _gate_body(x_ref, w1_ref, b1_ref, w2_ref, b2_ref, g_ref, gmax_ref):
    i = pl.program_id(0)
    h = jnp.maximum(x_ref[...] @ w1_ref[...] + b1_ref[...], 0.0)
    g = h @ w2_ref[...] + b2_ref[...]
    g_ref[...] = g
    bm = jnp.max(g)

    @pl.when(i == 0)
    def _():
        gmax_ref[...] = jnp.full((1, 1), bm)

    @pl.when(i > 0)
    def _():
        gmax_ref[...] = jnp.maximum(gmax_ref[...], bm)


def _gate(x, gW1, gb1, gW2, gb2):
    n = x.shape[0]
    return pl.pallas_call(
        _gate_body,
        grid=(n // BK,),
        in_specs=[
            pl.BlockSpec((BK, 64), lambda i: (i, 0)),
            pl.BlockSpec((64, 32), lambda i: (0, 0)),
            pl.BlockSpec((1, 32), lambda i: (0, 0)),
            pl.BlockSpec((32, 1), lambda i: (0, 0)),
            pl.BlockSpec((1, 1), lambda i: (0, 0)),
        ],
        out_specs=[
            pl.BlockSpec((BK, 1), lambda i: (i, 0)),
            pl.BlockSpec((1, 1), lambda i: (0, 0)),
        ],
        out_shape=[
            jax.ShapeDtypeStruct((n, 1), jnp.float32),
            jax.ShapeDtypeStruct((1, 1), jnp.float32),
        ],
    )(x, gW1, gb1, gW2, gb2)


def _sc_pool(x, gpad, spad, gmax16):
    n = x.shape[0]
    rows_per_tile = n // NW
    nchunk = rows_per_tile // CHUNK
    tail = rows_per_tile - nchunk * CHUNK
    tailpad = -(-tail // 16) * 16 if tail else 0

    mesh = plsc.VectorSubcoreMesh(
        core_axis_name="c", subcore_axis_name="s", num_cores=NC, num_subcores=NS
    )

    def body(
        x_hbm, g_hbm, s_hbm, gmax_hbm, out_hbm, xbuf, gbuf, sbuf, ebuf, gmv, acc
    ):
        wid = lax.axis_index("c") * NS + lax.axis_index("s")
        base = wid * rows_per_tile
        iota = lax.iota(jnp.int32, 16)
        onehot0 = (iota == 0).astype(jnp.float32)
        zeros = jnp.zeros((16,), jnp.float32)

        pltpu.sync_copy(gmax_hbm, gmv)

        def zbody(i, _):
            for d in range(ACC_W // 16):
                acc[i, pl.ds(16 * d, 16)] = zeros
            return 0

        lax.fori_loop(0, NUM_SEG, zbody, 0)

        def row(i):
            idxc = jnp.full((16,), i, jnp.int32)
            e_b = plsc.load_gather(ebuf, [idxc])
            s_b = plsc.load_gather(sbuf, [idxc])
            for d in range(4):
                xv = xbuf[i, pl.ds(16 * d, 16)]
                plsc.addupdate_scatter(acc, [s_b, iota + 16 * d], xv * e_b)
            plsc.addupdate_scatter(acc, [s_b, iota + 64], e_b * onehot0)

        def chunk_body(c, _):
            row0 = base + c * CHUNK
            pltpu.sync_copy(x_hbm.at[pl.ds(row0, CHUNK)], xbuf)
            pltpu.sync_copy(g_hbm.at[pl.ds(row0, CHUNK)], gbuf)
            pltpu.sync_copy(s_hbm.at[pl.ds(row0, CHUNK)], sbuf)
            for g0 in range(0, CHUNK, 16):
                ebuf[pl.ds(g0, 16)] = jnp.exp(gbuf[pl.ds(g0, 16)] - gmv[...])
            for i in range(CHUNK):
                row(i)
            return 0

        lax.fori_loop(0, nchunk, chunk_body, 0)

        if tail:
            row0 = base + nchunk * CHUNK
            pltpu.sync_copy(x_hbm.at[pl.ds(row0, tail)], xbuf.at[pl.ds(0, tail)])
            pltpu.sync_copy(g_hbm.at[pl.ds(row0, tailpad)], gbuf.at[pl.ds(0, tailpad)])
            pltpu.sync_copy(s_hbm.at[pl.ds(row0, tailpad)], sbuf.at[pl.ds(0, tailpad)])
            for g0 in range(0, tailpad, 16):
                ebuf[pl.ds(g0, 16)] = jnp.exp(gbuf[pl.ds(g0, 16)] - gmv[...])
            for i in range(tail):
                row(i)

        pltpu.sync_copy(acc, out_hbm.at[wid])

    run = pl.kernel(
        body,
        out_type=jax.ShapeDtypeStruct((NW, NUM_SEG, ACC_W), jnp.float32),
        mesh=mesh,
        compiler_params=pltpu.CompilerParams(
            use_tc_tiling_on_sc=False, needs_layout_passes=False
        ),
        scratch_types=[
            pltpu.VMEM((CHUNK, 64), jnp.float32),
            pltpu.VMEM((CHUNK,), jnp.float32),
            pltpu.VMEM((CHUNK,), jnp.int32),
            pltpu.VMEM((CHUNK,), jnp.float32),
            pltpu.VMEM((16,), jnp.float32),
            pltpu.VMEM((NUM_SEG, ACC_W), jnp.float32),
        ],
    )
    return run(x, gpad, spad, gmax16)


def _final_body(p_ref, w1_ref, b1_ref, w2_ref, b2_ref, out_ref):
    s = jnp.sum(p_ref[...], axis=0)
    num = s[:, :64]
    den = s[:, 64:65]
    hg = num / (den + 1e-16)
    h = jnp.maximum(hg @ w1_ref[...] + b1_ref[...], 0.0)
    out_ref[...] = h @ w2_ref[...] + b2_ref[...]


def _final(partials, mW1, mb1, mW2, mb2):
    return pl.pallas_call(
        _final_body,
        out_shape=jax.ShapeDtypeStruct((NUM_SEG, 1), jnp.float32),
    )(partials, mW1, mb1, mW2, mb2)


@jax.jit
def kernel(x, batch, gW1, gb1, gW2, gb2, mW1, mb1, mW2, mb2):
    g2d, gmax = _gate(x, gW1, gb1.reshape(1, -1), gW2, gb2.reshape(1, -1))
    g = g2d.reshape(-1)
    seg = batch.astype(jnp.int32)
    gpad = jnp.concatenate([g, jnp.zeros((16,), jnp.float32)])
    spad = jnp.concatenate([seg, jnp.zeros((16,), jnp.int32)])
    gmax16 = jnp.broadcast_to(gmax.reshape(1), (16,))
    partials = _sc_pool(x, gpad, spad, gmax16)
    return _final(partials, mW1, mb1.reshape(1, -1), mW2, mb2.reshape(1, -1))


# speedup vs baseline: 10.4377x; 1.7949x over previous
import jax
import jax.numpy as jnp
from jax import lax
from jax.experimental import pallas as pl
from jax.experimental.pallas import tpu as pltpu
from jax.experimental.pallas import tpu_sc as plsc

NUM_SEG = 1024
ACC_W = 80
NC, NS = 2, 16
NW = NC * NS
CHUNK = 512
BK = 4000


def _gate_body(x_ref, w1_ref, b1_ref, w2_ref, b2_ref, g_ref, gmax_ref):
    i = pl.program_id(0)
    h = jnp.maximum(x_ref[...] @ w1_ref[...] + b1_ref[...], 0.0)
    g = h @ w2_ref[...] + b2_ref[...]
    g_ref[...] = g
    bm = jnp.max(g)

    @pl.when(i == 0)
    def _():
        gmax_ref[...] = jnp.full((1, 1), bm)

    @pl.when(i > 0)
    def _():
        gmax_ref[...] = jnp.maximum(gmax_ref[...], bm)


def _gate(x, gW1, gb1, gW2, gb2):
    n = x.shape[0]
    return pl.pallas_call(
        _gate_body,
        grid=(n // BK,),
        in_specs=[
            pl.BlockSpec((BK, 64), lambda i: (i, 0)),
            pl.BlockSpec((64, 32), lambda i: (0, 0)),
            pl.BlockSpec((1, 32), lambda i: (0, 0)),
            pl.BlockSpec((32, 1), lambda i: (0, 0)),
            pl.BlockSpec((1, 1), lambda i: (0, 0)),
        ],
        out_specs=[
            pl.BlockSpec((BK, 1), lambda i: (i, 0)),
            pl.BlockSpec((1, 1), lambda i: (0, 0)),
        ],
        out_shape=[
            jax.ShapeDtypeStruct((n, 1), jnp.float32),
            jax.ShapeDtypeStruct((1, 1), jnp.float32),
        ],
    )(x, gW1, gb1, gW2, gb2)


def _sc_pool(x, gpad, spad, gmax16):
    n = x.shape[0]
    rows_per_tile = n // NW
    nchunk = rows_per_tile // CHUNK
    tail = rows_per_tile - nchunk * CHUNK
    tailpad = -(-tail // 16) * 16 if tail else 0
    tail_groups = tail // 16
    tail_rem = tail - tail_groups * 16

    mesh = plsc.VectorSubcoreMesh(
        core_axis_name="c", subcore_axis_name="s", num_cores=NC, num_subcores=NS
    )

    def body(
        x_hbm, g_hbm, s_hbm, gmax_hbm, out_hbm,
        xbuf, gbuf, sbuf, ebuf, gmv, acc, semx, semg, sems,
    ):
        wid = lax.axis_index("c") * NS + lax.axis_index("s")
        base = wid * rows_per_tile
        iota = lax.iota(jnp.int32, 16)
        onehot0 = (iota == 0).astype(jnp.float32)
        zf = jnp.zeros((16,), jnp.float32)
        zi = jnp.zeros((16,), jnp.int32)

        pltpu.sync_copy(gmax_hbm, gmv)

        def zbody(i, _):
            for d in range(ACC_W // 16):
                acc[i, pl.ds(16 * d, 16)] = zf
            return 0

        lax.fori_loop(0, NUM_SEG, zbody, 0)

        def flush(a, cur):
            for d in range(4):
                plsc.addupdate_scatter(acc, [cur, iota + 16 * d], a[d])
            plsc.addupdate_scatter(acc, [cur, iota + 64], a[4] * onehot0)

        def direct_row(i):
            idxc = jnp.full((16,), i, jnp.int32)
            e_b = plsc.load_gather(ebuf, [idxc])
            s_b = plsc.load_gather(sbuf, [idxc])
            for d in range(4):
                xv = xbuf[i, pl.ds(16 * d, 16)]
                plsc.addupdate_scatter(acc, [s_b, iota + 16 * d], xv * e_b)
            plsc.addupdate_scatter(acc, [s_b, iota + 64], e_b * onehot0)

        def group_body(g0, carry):
            a0, a1, a2, a3, ad, cur = carry
            seg16 = sbuf[pl.ds(g0, 16)]
            allsame = jnp.min((seg16 == cur).astype(jnp.int32))

            def fast(_):
                r0, r1, r2, r3, rd = a0, a1, a2, a3, ad
                for r in range(16):
                    idxc = jnp.full((16,), 1, jnp.int32) * (g0 + r)
                    e_b = plsc.load_gather(ebuf, [idxc])
                    r0 = r0 + xbuf[g0 + r, pl.ds(0, 16)] * e_b
                    r1 = r1 + xbuf[g0 + r, pl.ds(16, 16)] * e_b
                    r2 = r2 + xbuf[g0 + r, pl.ds(32, 16)] * e_b
                    r3 = r3 + xbuf[g0 + r, pl.ds(48, 16)] * e_b
                    rd = rd + e_b
                return (r0, r1, r2, r3, rd, cur)

            def slow(_):
                flush((a0, a1, a2, a3, ad), cur)
                for r in range(16):
                    direct_row(g0 + r)
                lastc = jnp.full((16,), 1, jnp.int32) * (g0 + 15)
                newcur = plsc.load_gather(sbuf, [lastc])
                return (zf, zf, zf, zf, zf, newcur)

            return lax.cond(allsame == 1, fast, slow, 0)

        def load_chunk(row0, nrows, npad):
            cx = pltpu.async_copy(
                x_hbm.at[pl.ds(row0, nrows)], xbuf.at[pl.ds(0, nrows)], semx
            )
            cg = pltpu.async_copy(
                g_hbm.at[pl.ds(row0, npad)], gbuf.at[pl.ds(0, npad)], semg
            )
            cs = pltpu.async_copy(
                s_hbm.at[pl.ds(row0, npad)], sbuf.at[pl.ds(0, npad)], sems
            )
            cx.wait()
            cg.wait()
            cs.wait()

        def chunk_body(c, carry):
            load_chunk(base + c * CHUNK, CHUNK, CHUNK)
            for g0 in range(0, CHUNK, 16):
                ebuf[pl.ds(g0, 16)] = jnp.exp(gbuf[pl.ds(g0, 16)] - gmv[...])
            return lax.fori_loop(
                0, CHUNK // 16, lambda i, car: group_body(i * 16, car), carry
            )

        carry = (zf, zf, zf, zf, zf, zi)
        carry = lax.fori_loop(0, nchunk, chunk_body, carry)

        if tail:
            load_chunk(base + nchunk * CHUNK, tail, tailpad)
            for g0 in range(0, tailpad, 16):
                ebuf[pl.ds(g0, 16)] = jnp.exp(gbuf[pl.ds(g0, 16)] - gmv[...])
            carry = lax.fori_loop(
                0, tail_groups, lambda i, car: group_body(i * 16, car), carry
            )
            for r in range(tail_rem):
                direct_row(tail_groups * 16 + r)

        flush(carry[:5], carry[5])

        pltpu.sync_copy(acc, out_hbm.at[wid])

    run = pl.kernel(
        body,
        out_type=jax.ShapeDtypeStruct((NW, NUM_SEG, ACC_W), jnp.float32),
        mesh=mesh,
        compiler_params=pltpu.CompilerParams(
            use_tc_tiling_on_sc=False, needs_layout_passes=False
        ),
        scratch_types=[
            pltpu.VMEM((CHUNK, 64), jnp.float32),
            pltpu.VMEM((CHUNK,), jnp.float32),
            pltpu.VMEM((CHUNK,), jnp.int32),
            pltpu.VMEM((CHUNK,), jnp.float32),
            pltpu.VMEM((16,), jnp.float32),
            pltpu.VMEM((NUM_SEG, ACC_W), jnp.float32),
            pltpu.SemaphoreType.DMA,
            pltpu.SemaphoreType.DMA,
            pltpu.SemaphoreType.DMA,
        ],
    )
    return run(x, gpad, spad, gmax16)


def _final_body(p_ref, w1_ref, b1_ref, w2_ref, b2_ref, out_ref):
    s = jnp.sum(p_ref[...], axis=0)
    num = s[:, :64]
    den = s[:, 64:65]
    hg = num / (den + 1e-16)
    h = jnp.maximum(hg @ w1_ref[...] + b1_ref[...], 0.0)
    out_ref[...] = h @ w2_ref[...] + b2_ref[...]


def _final(partials, mW1, mb1, mW2, mb2):
    return pl.pallas_call(
        _final_body,
        out_shape=jax.ShapeDtypeStruct((NUM_SEG, 1), jnp.float32),
    )(partials, mW1, mb1, mW2, mb2)


@jax.jit
def kernel(x, batch, gW1, gb1, gW2, gb2, mW1, mb1, mW2, mb2):
    g2d, gmax = _gate(x, gW1, gb1.reshape(1, -1), gW2, gb2.reshape(1, -1))
    g = g2d.reshape(-1)
    seg = batch.astype(jnp.int32)
    gpad = jnp.concatenate([g, jnp.zeros((16,), jnp.float32)])
    spad = jnp.concatenate([seg, jnp.zeros((16,), jnp.int32)])
    gmax16 = jnp.broadcast_to(gmax.reshape(1), (16,))
    partials = _sc_pool(x, gpad, spad, gmax16)
    return _final(partials, mW1, mb1.reshape(1, -1), mW2, mb2.reshape(1, -1))


# speedup vs baseline: 11.1145x; 1.0648x over previous
import jax
import jax.numpy as jnp
from jax import lax
from jax.experimental import pallas as pl
from jax.experimental.pallas import tpu as pltpu
from jax.experimental.pallas import tpu_sc as plsc

NUM_SEG = 1024
ACC_W = 80
NC, NS = 2, 16
NW = NC * NS
CHUNK = 512
BK = 16000


def _gate_body(x_ref, w1_ref, b1_ref, w2_ref, b2_ref, g_ref, gmax_ref):
    i = pl.program_id(0)
    h = jnp.maximum(x_ref[...] @ w1_ref[...] + b1_ref[...], 0.0)
    g = jnp.sum(h * w2_ref[...], axis=1, keepdims=True) + b2_ref[...]
    g_ref[...] = g
    bm = jnp.max(g)

    @pl.when(i == 0)
    def _():
        gmax_ref[...] = jnp.full((1, 1), bm)

    @pl.when(i > 0)
    def _():
        gmax_ref[...] = jnp.maximum(gmax_ref[...], bm)


def _gate(x, gW1, gb1, gW2, gb2):
    n = x.shape[0]
    return pl.pallas_call(
        _gate_body,
        grid=(n // BK,),
        in_specs=[
            pl.BlockSpec((BK, 64), lambda i: (i, 0)),
            pl.BlockSpec((64, 32), lambda i: (0, 0)),
            pl.BlockSpec((1, 32), lambda i: (0, 0)),
            pl.BlockSpec((1, 32), lambda i: (0, 0)),
            pl.BlockSpec((1, 1), lambda i: (0, 0)),
        ],
        out_specs=[
            pl.BlockSpec((BK, 1), lambda i: (i, 0)),
            pl.BlockSpec((1, 1), lambda i: (0, 0)),
        ],
        out_shape=[
            jax.ShapeDtypeStruct((n, 1), jnp.float32),
            jax.ShapeDtypeStruct((1, 1), jnp.float32),
        ],
    )(x, gW1, gb1, gW2, gb2)


def _sc_pool(x, gpad, spad, gmax16):
    n = x.shape[0]
    rows_per_tile = n // NW
    nchunk = rows_per_tile // CHUNK
    tail = rows_per_tile - nchunk * CHUNK
    tailpad = -(-tail // 16) * 16 if tail else 0
    tail_groups = tail // 16
    tail_rem = tail - tail_groups * 16

    mesh = plsc.VectorSubcoreMesh(
        core_axis_name="c", subcore_axis_name="s", num_cores=NC, num_subcores=NS
    )

    def body(
        x_hbm, g_hbm, s_hbm, gmax_hbm, out_hbm,
        xbuf, gbuf, sbuf, ebuf, gmv, acc, semx, semg, sems,
    ):
        wid = lax.axis_index("c") * NS + lax.axis_index("s")
        base = wid * rows_per_tile
        iota = lax.iota(jnp.int32, 16)
        onehot0 = (iota == 0).astype(jnp.float32)
        zf = jnp.zeros((16,), jnp.float32)
        zi = jnp.zeros((16,), jnp.int32)

        pltpu.sync_copy(gmax_hbm, gmv)

        def zbody(i, _):
            for d in range(ACC_W // 16):
                acc[i, pl.ds(16 * d, 16)] = zf
            return 0

        lax.fori_loop(0, NUM_SEG, zbody, 0)

        def flush(a, cur):
            for d in range(4):
                plsc.addupdate_scatter(acc, [cur, iota + 16 * d], a[d])
            plsc.addupdate_scatter(acc, [cur, iota + 64], a[4] * onehot0)

        def direct_row(i):
            idxc = jnp.full((16,), i, jnp.int32)
            e_b = plsc.load_gather(ebuf, [idxc])
            s_b = plsc.load_gather(sbuf, [idxc])
            for d in range(4):
                xv = xbuf[i, pl.ds(16 * d, 16)]
                plsc.addupdate_scatter(acc, [s_b, iota + 16 * d], xv * e_b)
            plsc.addupdate_scatter(acc, [s_b, iota + 64], e_b * onehot0)

        def group_body(g0, carry):
            a0, a1, a2, a3, ad, cur = carry
            seg16 = sbuf[pl.ds(g0, 16)]
            allsame = jnp.min((seg16 == cur).astype(jnp.int32))

            def fast(_):
                r0, r1, r2, r3, rd = a0, a1, a2, a3, ad
                for r in range(16):
                    idxc = jnp.full((16,), 1, jnp.int32) * (g0 + r)
                    e_b = plsc.load_gather(ebuf, [idxc])
                    r0 = r0 + xbuf[g0 + r, pl.ds(0, 16)] * e_b
                    r1 = r1 + xbuf[g0 + r, pl.ds(16, 16)] * e_b
                    r2 = r2 + xbuf[g0 + r, pl.ds(32, 16)] * e_b
                    r3 = r3 + xbuf[g0 + r, pl.ds(48, 16)] * e_b
                    rd = rd + e_b
                return (r0, r1, r2, r3, rd, cur)

            def slow(_):
                flush((a0, a1, a2, a3, ad), cur)
                for r in range(16):
                    direct_row(g0 + r)
                lastc = jnp.full((16,), 1, jnp.int32) * (g0 + 15)
                newcur = plsc.load_gather(sbuf, [lastc])
                return (zf, zf, zf, zf, zf, newcur)

            return lax.cond(allsame == 1, fast, slow, 0)

        def load_chunk(row0, nrows, npad):
            cx = pltpu.async_copy(
                x_hbm.at[pl.ds(row0, nrows)], xbuf.at[pl.ds(0, nrows)], semx
            )
            cg = pltpu.async_copy(
                g_hbm.at[pl.ds(row0, npad)], gbuf.at[pl.ds(0, npad)], semg
            )
            cs = pltpu.async_copy(
                s_hbm.at[pl.ds(row0, npad)], sbuf.at[pl.ds(0, npad)], sems
            )
            cx.wait()
            cg.wait()
            cs.wait()

        def chunk_body(c, carry):
            load_chunk(base + c * CHUNK, CHUNK, CHUNK)
            for g0 in range(0, CHUNK, 16):
                ebuf[pl.ds(g0, 16)] = jnp.exp(gbuf[pl.ds(g0, 16)] - gmv[...])
            return lax.fori_loop(
                0, CHUNK // 16, lambda i, car: group_body(i * 16, car), carry
            )

        carry = (zf, zf, zf, zf, zf, zi)
        carry = lax.fori_loop(0, nchunk, chunk_body, carry)

        if tail:
            load_chunk(base + nchunk * CHUNK, tail, tailpad)
            for g0 in range(0, tailpad, 16):
                ebuf[pl.ds(g0, 16)] = jnp.exp(gbuf[pl.ds(g0, 16)] - gmv[...])
            carry = lax.fori_loop(
                0, tail_groups, lambda i, car: group_body(i * 16, car), carry
            )
            for r in range(tail_rem):
                direct_row(tail_groups * 16 + r)

        flush(carry[:5], carry[5])

        pltpu.sync_copy(acc, out_hbm.at[wid])

    run = pl.kernel(
        body,
        out_type=jax.ShapeDtypeStruct((NW, NUM_SEG, ACC_W), jnp.float32),
        mesh=mesh,
        compiler_params=pltpu.CompilerParams(
            use_tc_tiling_on_sc=False, needs_layout_passes=False
        ),
        scratch_types=[
            pltpu.VMEM((CHUNK, 64), jnp.float32),
            pltpu.VMEM((CHUNK,), jnp.float32),
            pltpu.VMEM((CHUNK,), jnp.int32),
            pltpu.VMEM((CHUNK,), jnp.float32),
            pltpu.VMEM((16,), jnp.float32),
            pltpu.VMEM((NUM_SEG, ACC_W), jnp.float32),
            pltpu.SemaphoreType.DMA,
            pltpu.SemaphoreType.DMA,
            pltpu.SemaphoreType.DMA,
        ],
    )
    return run(x, gpad, spad, gmax16)


def _final_body(p_ref, w1_ref, b1_ref, w2_ref, b2_ref, out_ref):
    s = jnp.sum(p_ref[...], axis=0)
    num = s[:, :64]
    den = s[:, 64:65]
    hg = num / (den + 1e-16)
    h = jnp.maximum(hg @ w1_ref[...] + b1_ref[...], 0.0)
    out_ref[...] = h @ w2_ref[...] + b2_ref[...]


def _final(partials, mW1, mb1, mW2, mb2):
    return pl.pallas_call(
        _final_body,
        out_shape=jax.ShapeDtypeStruct((NUM_SEG, 1), jnp.float32),
    )(partials, mW1, mb1, mW2, mb2)


@jax.jit
def kernel(x, batch, gW1, gb1, gW2, gb2, mW1, mb1, mW2, mb2):
    g2d, gmax = _gate(x, gW1, gb1.reshape(1, -1), gW2.reshape(1, -1), gb2.reshape(1, -1))
    g = g2d.reshape(-1)
    seg = batch.astype(jnp.int32)
    gpad = jnp.concatenate([g, jnp.zeros((16,), jnp.float32)])
    spad = jnp.concatenate([seg, jnp.zeros((16,), jnp.int32)])
    gmax16 = jnp.broadcast_to(gmax.reshape(1), (16,))
    partials = _sc_pool(x, gpad, spad, gmax16)
    return _final(partials, mW1, mb1.reshape(1, -1), mW2, mb2.reshape(1, -1))
